# Initial kernel scaffold; baseline (speedup 1.0000x reference)
import functools

import jax
import jax.numpy as jnp
from jax import lax
from jax.experimental import pallas as pl
from jax.experimental.pallas import tpu as pltpu
from jax.experimental.pallas import tpu_sc as plsc

N = 10000
E = 160000
D = 256
H = 128
NG = 128
DOUT = 128
NSUB = 16
EPS = E // NSUB
RPS = N // NSUB
CHUNK = 128
NFULL = EPS // CHUNK
TAIL = EPS - NFULL * CHUNK

_MESH = plsc.VectorSubcoreMesh(core_axis_name="c", subcore_axis_name="s")


@functools.partial(
    pl.kernel,
    out_type=jax.ShapeDtypeStruct((NSUB, N), jnp.float32),
    mesh=_MESH,
    scratch_types=[
        pltpu.VMEM((EPS,), jnp.int32),
        pltpu.VMEM((EPS,), jnp.float32),
        pltpu.VMEM((N,), jnp.float32),
    ],
)
def _deg_kernel(dst_hbm, w_hbm, out_hbm, didx_v, wv_v, acc_v):
    c = lax.axis_index("c")
    s = lax.axis_index("s")

    @pl.when(c == 0)
    def _():
        zero = jnp.zeros((16,), jnp.float32)

        def zbody(i, carry):
            acc_v[pl.ds(i * 16, 16)] = zero
            return carry

        lax.fori_loop(0, N // 16, zbody, 0)
        base = s * EPS
        pltpu.sync_copy(dst_hbm.at[pl.ds(base, EPS)], didx_v)
        pltpu.sync_copy(w_hbm.at[pl.ds(base, EPS)], wv_v)

        def ebody(i, carry):
            idx = didx_v[pl.ds(i * 16, 16)]
            vals = wv_v[pl.ds(i * 16, 16)]
            plsc.addupdate_scatter(acc_v, [idx], vals)
            return carry

        lax.fori_loop(0, EPS // 16, ebody, 0)
        pltpu.sync_copy(acc_v, out_hbm.at[s])


@functools.partial(
    pl.kernel,
    out_type=(
        jax.ShapeDtypeStruct((N, H), jnp.float32),
        jax.ShapeDtypeStruct((N, H), jnp.float32),
    ),
    mesh=_MESH,
    scratch_types=[
        pltpu.VMEM((CHUNK,), jnp.int32),
        pltpu.VMEM((CHUNK,), jnp.int32),
        pltpu.VMEM((CHUNK,), jnp.float32),
        pltpu.VMEM((CHUNK, H), jnp.float32),
        pltpu.VMEM((TAIL,), jnp.int32),
        pltpu.VMEM((TAIL,), jnp.int32),
        pltpu.VMEM((TAIL,), jnp.float32),
        pltpu.VMEM((TAIL, H), jnp.float32),
        pltpu.VMEM_SHARED((N, H), jnp.float32),
        pltpu.SemaphoreType.DMA,
    ],
)
def _agg_kernel(xa_hbm, xb_hbm, src_hbm, dst_hbm, w_hbm, oa_hbm, ob_hbm,
                sidx_v, didx_v, wv_v, rows_v,
                sidx_t, didx_t, wv_t, rows_t,
                acc_sp, sem):
    c = lax.axis_index("c")
    s = lax.axis_index("s")
    zero = jnp.zeros((16,), jnp.float32)

    def zrow(i, carry):
        for f in range(H // 16):
            rows_v[i, pl.ds(f * 16, 16)] = zero
        return carry

    lax.fori_loop(0, CHUNK, zrow, 0)
    for k in range(5):
        pltpu.sync_copy(rows_v.at[pl.ds(0, 125)],
                        acc_sp.at[pl.ds(s * RPS + k * 125, 125)])
    plsc.subcore_barrier()

    ebase = s * EPS

    def run(x_hbm):
        def chunk(si, di, wr, rows, off, k):
            pltpu.sync_copy(src_hbm.at[pl.ds(off, k)], si)
            pltpu.sync_copy(dst_hbm.at[pl.ds(off, k)], di)
            pltpu.sync_copy(w_hbm.at[pl.ds(off, k)], wr)
            pltpu.async_copy(x_hbm.at[si], rows, sem).wait()

            def mul(j, carry):
                wb = jnp.broadcast_to(wr[j], (16,))
                for f in range(H // 16):
                    rows[j, pl.ds(f * 16, 16)] = rows[j, pl.ds(f * 16, 16)] * wb
                return carry

            lax.fori_loop(0, k, mul, 0)
            pltpu.sync_copy(rows, acc_sp.at[di], add=True)

        def body(i, carry):
            chunk(sidx_v, didx_v, wv_v, rows_v, ebase + i * CHUNK, CHUNK)
            return carry

        lax.fori_loop(0, NFULL, body, 0)
        chunk(sidx_t, didx_t, wv_t, rows_t, ebase + NFULL * CHUNK, TAIL)

    @pl.when(c == 0)
    def _():
        run(xa_hbm)

    @pl.when(c == 1)
    def _():
        run(xb_hbm)

    plsc.subcore_barrier()

    @pl.when(c == 0)
    def _():
        pltpu.sync_copy(acc_sp.at[pl.ds(s * RPS, RPS)],
                        oa_hbm.at[pl.ds(s * RPS, RPS)])

    @pl.when(c == 1)
    def _():
        pltpu.sync_copy(acc_sp.at[pl.ds(s * RPS, RPS)],
                        ob_hbm.at[pl.ds(s * RPS, RPS)])


R = 1000
GRID = N // R


def _dinv_from(deg_ref):
    deg = jnp.sum(deg_ref[...], axis=0) + 1.0
    return jnp.where(deg > 0, lax.rsqrt(deg), 0.0)


def _mm1_body(x_ref, w_ref, deg_ref, xa_ref, xb_ref):
    dinv = _dinv_from(deg_ref)
    xw = jnp.dot(x_ref[...], w_ref[...], preferred_element_type=jnp.float32)
    xws = xw * dinv[:, None]
    xa_ref[...] = xws[:, :H]
    xb_ref[...] = xws[:, H:]


def _mm1(x, W, deg16):
    return pl.pallas_call(
        _mm1_body,
        grid=(GRID,),
        in_specs=[
            pl.BlockSpec((R, D), lambda i: (i, 0)),
            pl.BlockSpec((D, D), lambda i: (0, 0)),
            pl.BlockSpec((NSUB, R), lambda i: (0, i)),
        ],
        out_specs=[
            pl.BlockSpec((R, H), lambda i: (i, 0)),
            pl.BlockSpec((R, H), lambda i: (i, 0)),
        ],
        out_shape=[jax.ShapeDtypeStruct((N, H), jnp.float32)] * 2,
    )(x, W, deg16)


def _mid_body(aa_ref, ab_ref, xa_ref, xb_ref, deg_ref, b_ref, w_ref,
              oa_ref, ob_ref):
    dinv = _dinv_from(deg_ref)
    acc = jnp.concatenate([aa_ref[...], ab_ref[...]], axis=1)
    xws = jnp.concatenate([xa_ref[...], xb_ref[...]], axis=1)
    h = jnp.maximum(dinv[:, None] * (acc + xws) + b_ref[...], 0.0)
    hw = jnp.dot(h, w_ref[...], preferred_element_type=jnp.float32)
    hws = hw * dinv[:, None]
    oa_ref[...] = hws[:, :H]
    ob_ref[...] = hws[:, H:]


def _mid(aa, ab, xa, xb, deg16, b, W):
    return pl.pallas_call(
        _mid_body,
        grid=(GRID,),
        in_specs=[
            pl.BlockSpec((R, H), lambda i: (i, 0)),
            pl.BlockSpec((R, H), lambda i: (i, 0)),
            pl.BlockSpec((R, H), lambda i: (i, 0)),
            pl.BlockSpec((R, H), lambda i: (i, 0)),
            pl.BlockSpec((NSUB, R), lambda i: (0, i)),
            pl.BlockSpec((1, D), lambda i: (0, 0)),
            pl.BlockSpec((D, D), lambda i: (0, 0)),
        ],
        out_specs=[
            pl.BlockSpec((R, H), lambda i: (i, 0)),
            pl.BlockSpec((R, H), lambda i: (i, 0)),
        ],
        out_shape=[jax.ShapeDtypeStruct((N, H), jnp.float32)] * 2,
    )(aa, ab, xa, xb, deg16, b, W)


def _final_body(aa_ref, ab_ref, xa_ref, xb_ref, deg_ref, b3_ref, batch_ref,
                wl1_ref, bl1_ref, wl2_ref, bl2_ref, out_ref, sums, cnt):
    i = pl.program_id(0)
    dinv = _dinv_from(deg_ref)
    acc = jnp.concatenate([aa_ref[...], ab_ref[...]], axis=1)
    xws = jnp.concatenate([xa_ref[...], xb_ref[...]], axis=1)
    h3 = jnp.maximum(dinv[:, None] * (acc + xws) + b3_ref[...], 0.0)
    seg = batch_ref[0, 0, :]
    onehot = (seg[:, None]
              == lax.broadcasted_iota(jnp.int32, (1, NG), 1)).astype(jnp.float32)
    blk_sums = lax.dot_general(onehot, h3, (((0,), (0,)), ((), ())),
                               preferred_element_type=jnp.float32)
    ones = jnp.ones((R, 1), jnp.float32)
    blk_cnt = lax.dot_general(onehot, ones, (((0,), (0,)), ((), ())),
                              preferred_element_type=jnp.float32)

    @pl.when(i == 0)
    def _():
        sums[...] = jnp.zeros_like(sums)
        cnt[...] = jnp.zeros_like(cnt)

    sums[...] += blk_sums
    cnt[...] += blk_cnt

    @pl.when(i == pl.num_programs(0) - 1)
    def _():
        pooled = sums[...] / jnp.maximum(cnt[...], 1.0)
        l1 = jnp.maximum(
            jnp.dot(pooled, wl1_ref[...], preferred_element_type=jnp.float32)
            + bl1_ref[...], 0.0)
        o = (jnp.dot(l1, wl2_ref[...], preferred_element_type=jnp.float32)
             + bl2_ref[...])
        m = jnp.max(o, axis=-1, keepdims=True)
        lse = m + jnp.log(jnp.sum(jnp.exp(o - m), axis=-1, keepdims=True))
        out_ref[...] = o - lse


def _final(aa, ab, xa, xb, deg16, b3, batch3, Wl1, bl1, Wl2, bl2):
    return pl.pallas_call(
        _final_body,
        grid=(GRID,),
        in_specs=[
            pl.BlockSpec((R, H), lambda i: (i, 0)),
            pl.BlockSpec((R, H), lambda i: (i, 0)),
            pl.BlockSpec((R, H), lambda i: (i, 0)),
            pl.BlockSpec((R, H), lambda i: (i, 0)),
            pl.BlockSpec((NSUB, R), lambda i: (0, i)),
            pl.BlockSpec((1, D), lambda i: (0, 0)),
            pl.BlockSpec((1, 1, R), lambda i: (i, 0, 0)),
            pl.BlockSpec((D, D), lambda i: (0, 0)),
            pl.BlockSpec((1, D), lambda i: (0, 0)),
            pl.BlockSpec((D, DOUT), lambda i: (0, 0)),
            pl.BlockSpec((1, DOUT), lambda i: (0, 0)),
        ],
        out_specs=pl.BlockSpec((NG, DOUT), lambda i: (0, 0)),
        out_shape=jax.ShapeDtypeStruct((NG, DOUT), jnp.float32),
        scratch_shapes=[
            pltpu.VMEM((NG, D), jnp.float32),
            pltpu.VMEM((NG, 1), jnp.float32),
        ],
    )(aa, ab, xa, xb, deg16, b3, batch3, Wl1, bl1, Wl2, bl2)


def kernel(x, edge_index, edge_attr, batch, W1, b1, W2, b2, W3, b3,
           Wl1, bl1, Wl2, bl2):
    src = edge_index[0]
    dst = edge_index[1]
    deg16 = _deg_kernel(dst, edge_attr)
    xa, xb = _mm1(x, W1, deg16)
    aa, ab = _agg_kernel(xa, xb, src, dst, edge_attr)
    xa, xb = _mid(aa, ab, xa, xb, deg16, b1.reshape(1, D), W2)
    aa, ab = _agg_kernel(xa, xb, src, dst, edge_attr)
    xa, xb = _mid(aa, ab, xa, xb, deg16, b2.reshape(1, D), W3)
    aa, ab = _agg_kernel(xa, xb, src, dst, edge_attr)
    return _final(aa, ab, xa, xb, deg16, b3.reshape(1, D),
                  batch.reshape(GRID, 1, R), Wl1, bl1.reshape(1, D),
                  Wl2, bl2.reshape(1, DOUT))


# speedup vs baseline: 5.6380x; 5.6380x over previous
import functools

import jax
import jax.numpy as jnp
from jax import lax
from jax.experimental import pallas as pl
from jax.experimental.pallas import tpu as pltpu
from jax.experimental.pallas import tpu_sc as plsc

N = 10000
NP = 10240
E = 160000
D = 256
H = 128
NG = 128
DOUT = 128
NSUB = 16
EPS = E // NSUB
RPS = NP // NSUB
CHUNK = 128
NFULL = EPS // CHUNK
TAIL = EPS - NFULL * CHUNK

_MESH = plsc.VectorSubcoreMesh(core_axis_name="c", subcore_axis_name="s",
                               num_cores=2, num_subcores=NSUB)


def _make_agg(interpret=False):
    return functools.partial(
        pl.kernel,
        out_type=(
            jax.ShapeDtypeStruct((NP, H), jnp.float32),
            jax.ShapeDtypeStruct((NP, H), jnp.float32),
        ),
        mesh=_MESH,
        scratch_types=[
            pltpu.VMEM((CHUNK,), jnp.int32),
            pltpu.VMEM((CHUNK,), jnp.int32),
            pltpu.VMEM((CHUNK,), jnp.float32),
            pltpu.VMEM((CHUNK, H), jnp.float32),
            pltpu.VMEM((TAIL,), jnp.int32),
            pltpu.VMEM((TAIL,), jnp.int32),
            pltpu.VMEM((TAIL,), jnp.float32),
            pltpu.VMEM((TAIL, H), jnp.float32),
            pltpu.VMEM_SHARED((NP, H), jnp.float32),
            pltpu.SemaphoreType.DMA,
        ],
        interpret=interpret,
    )(_agg_body)


def _agg_body(xa_hbm, xb_hbm, src_hbm, dst_hbm, w_hbm, oa_hbm, ob_hbm,
                sidx_v, didx_v, wv_v, rows_v,
                sidx_t, didx_t, wv_t, rows_t,
                acc_sp, sem):
    c = lax.axis_index("c")
    s = lax.axis_index("s")
    zero = jnp.zeros((16,), jnp.float32)

    def zrow(i, carry):
        for f in range(H // 16):
            rows_v[i, pl.ds(f * 16, 16)] = zero
        return carry

    lax.fori_loop(0, CHUNK, zrow, 0)
    for k in range(5):
        pltpu.sync_copy(rows_v,
                        acc_sp.at[pl.ds(s * RPS + k * CHUNK, CHUNK)])
    plsc.subcore_barrier()

    ebase = s * EPS

    def run(x_hbm):
        def chunk(si, di, wr, rows, off, k):
            pltpu.sync_copy(src_hbm.at[pl.ds(off, k)], si)
            pltpu.sync_copy(dst_hbm.at[pl.ds(off, k)], di)
            pltpu.sync_copy(w_hbm.at[pl.ds(off, k)], wr)
            pltpu.async_copy(x_hbm.at[si], rows, sem).wait()

            def mul(g, carry):
                w16 = wr[pl.ds(g * 16, 16)]
                base16 = g * 16
                for t in range(16):
                    wb = jnp.broadcast_to(w16[t], (16,))
                    j = base16 + t
                    for f in range(H // 16):
                        rows[j, pl.ds(f * 16, 16)] = (
                            rows[j, pl.ds(f * 16, 16)] * wb)
                return carry

            lax.fori_loop(0, k // 16, mul, 0)
            pltpu.sync_copy(rows, acc_sp.at[di], add=True)

        def body(i, carry):
            chunk(sidx_v, didx_v, wv_v, rows_v, ebase + i * CHUNK, CHUNK)
            return carry

        lax.fori_loop(0, NFULL, body, 0)
        chunk(sidx_t, didx_t, wv_t, rows_t, ebase + NFULL * CHUNK, TAIL)

    @pl.when(c == 0)
    def _():
        run(xa_hbm)

    @pl.when(c == 1)
    def _():
        run(xb_hbm)

    plsc.subcore_barrier()

    @pl.when(c == 0)
    def _():
        pltpu.sync_copy(acc_sp.at[pl.ds(s * RPS, RPS)],
                        oa_hbm.at[pl.ds(s * RPS, RPS)])

    @pl.when(c == 1)
    def _():
        pltpu.sync_copy(acc_sp.at[pl.ds(s * RPS, RPS)],
                        ob_hbm.at[pl.ds(s * RPS, RPS)])


R = 1024
GRID = NP // R


def _dinv_from(deg_ref):
    deg = deg_ref[...][:, :1] + 1.0
    return jnp.where(deg > 0, lax.rsqrt(deg), 0.0)


def _mm1_body(x_ref, w_ref, deg_ref, xa_ref, xb_ref):
    dinv = _dinv_from(deg_ref)
    xw = jnp.dot(x_ref[...], w_ref[...], preferred_element_type=jnp.float32)
    xws = xw * dinv
    xa_ref[...] = xws[:, :H]
    xb_ref[...] = xws[:, H:]


def _mm1(x, W, deg16):
    return pl.pallas_call(
        _mm1_body,
        grid=(GRID,),
        in_specs=[
            pl.BlockSpec((R, D), lambda i: (i, 0)),
            pl.BlockSpec((D, D), lambda i: (0, 0)),
            pl.BlockSpec((R, H), lambda i: (i, 0)),
        ],
        out_specs=[
            pl.BlockSpec((R, H), lambda i: (i, 0)),
            pl.BlockSpec((R, H), lambda i: (i, 0)),
        ],
        out_shape=[jax.ShapeDtypeStruct((NP, H), jnp.float32)] * 2,
    )(x, W, deg16)


def _mid_body(aa_ref, ab_ref, xa_ref, xb_ref, deg_ref, b_ref, w_ref,
              oa_ref, ob_ref):
    dinv = _dinv_from(deg_ref)
    acc = jnp.concatenate([aa_ref[...], ab_ref[...]], axis=1)
    xws = jnp.concatenate([xa_ref[...], xb_ref[...]], axis=1)
    h = jnp.maximum(dinv * (acc + xws) + b_ref[...], 0.0)
    hw = jnp.dot(h, w_ref[...], preferred_element_type=jnp.float32)
    hws = hw * dinv
    oa_ref[...] = hws[:, :H]
    ob_ref[...] = hws[:, H:]


def _mid(aa, ab, xa, xb, deg16, b, W):
    return pl.pallas_call(
        _mid_body,
        grid=(GRID,),
        in_specs=[
            pl.BlockSpec((R, H), lambda i: (i, 0)),
            pl.BlockSpec((R, H), lambda i: (i, 0)),
            pl.BlockSpec((R, H), lambda i: (i, 0)),
            pl.BlockSpec((R, H), lambda i: (i, 0)),
            pl.BlockSpec((R, H), lambda i: (i, 0)),
            pl.BlockSpec((1, D), lambda i: (0, 0)),
            pl.BlockSpec((D, D), lambda i: (0, 0)),
        ],
        out_specs=[
            pl.BlockSpec((R, H), lambda i: (i, 0)),
            pl.BlockSpec((R, H), lambda i: (i, 0)),
        ],
        out_shape=[jax.ShapeDtypeStruct((NP, H), jnp.float32)] * 2,
    )(aa, ab, xa, xb, deg16, b, W)


def _final_body(aa_ref, ab_ref, xa_ref, xb_ref, deg_ref, b3_ref, batch_ref,
                wl1_ref, bl1_ref, wl2_ref, bl2_ref, out_ref, sums, cnt):
    i = pl.program_id(0)
    dinv = _dinv_from(deg_ref)
    acc = jnp.concatenate([aa_ref[...], ab_ref[...]], axis=1)
    xws = jnp.concatenate([xa_ref[...], xb_ref[...]], axis=1)
    h3 = jnp.maximum(dinv * (acc + xws) + b3_ref[...], 0.0)
    seg = batch_ref[0, 0, :]
    onehot = (seg[:, None]
              == lax.broadcasted_iota(jnp.int32, (1, NG), 1)).astype(jnp.float32)
    blk_sums = lax.dot_general(onehot, h3, (((0,), (0,)), ((), ())),
                               preferred_element_type=jnp.float32)
    ones = jnp.ones((R, 1), jnp.float32)
    blk_cnt = lax.dot_general(onehot, ones, (((0,), (0,)), ((), ())),
                              preferred_element_type=jnp.float32)

    @pl.when(i == 0)
    def _():
        sums[...] = jnp.zeros_like(sums)
        cnt[...] = jnp.zeros_like(cnt)

    sums[...] += blk_sums
    cnt[...] += blk_cnt

    @pl.when(i == pl.num_programs(0) - 1)
    def _():
        pooled = sums[...] / jnp.maximum(cnt[...], 1.0)
        l1 = jnp.maximum(
            jnp.dot(pooled, wl1_ref[...], preferred_element_type=jnp.float32)
            + bl1_ref[...], 0.0)
        o = (jnp.dot(l1, wl2_ref[...], preferred_element_type=jnp.float32)
             + bl2_ref[...])
        m = jnp.max(o, axis=-1, keepdims=True)
        lse = m + jnp.log(jnp.sum(jnp.exp(o - m), axis=-1, keepdims=True))
        out_ref[...] = o - lse


def _final(aa, ab, xa, xb, deg16, b3, batch3, Wl1, bl1, Wl2, bl2):
    return pl.pallas_call(
        _final_body,
        grid=(GRID,),
        in_specs=[
            pl.BlockSpec((R, H), lambda i: (i, 0)),
            pl.BlockSpec((R, H), lambda i: (i, 0)),
            pl.BlockSpec((R, H), lambda i: (i, 0)),
            pl.BlockSpec((R, H), lambda i: (i, 0)),
            pl.BlockSpec((R, H), lambda i: (i, 0)),
            pl.BlockSpec((1, D), lambda i: (0, 0)),
            pl.BlockSpec((1, 1, R), lambda i: (i, 0, 0)),
            pl.BlockSpec((D, D), lambda i: (0, 0)),
            pl.BlockSpec((1, D), lambda i: (0, 0)),
            pl.BlockSpec((D, DOUT), lambda i: (0, 0)),
            pl.BlockSpec((1, DOUT), lambda i: (0, 0)),
        ],
        out_specs=pl.BlockSpec((NG, DOUT), lambda i: (0, 0)),
        out_shape=jax.ShapeDtypeStruct((NG, DOUT), jnp.float32),
        scratch_shapes=[
            pltpu.VMEM((NG, D), jnp.float32),
            pltpu.VMEM((NG, 1), jnp.float32),
        ],
    )(aa, ab, xa, xb, deg16, b3, batch3, Wl1, bl1, Wl2, bl2)


_agg_kernel = _make_agg()


def kernel(x, edge_index, edge_attr, batch, W1, b1, W2, b2, W3, b3,
           Wl1, bl1, Wl2, bl2):
    src = edge_index[0]
    dst = edge_index[1]
    x = jnp.pad(x, ((0, NP - N), (0, 0)))
    batch = jnp.pad(batch, (0, NP - N), constant_values=NG)
    ones = jnp.ones((NP, H), jnp.float32)
    deg16, _unused = _agg_kernel(ones, ones, src, dst, edge_attr)
    xa, xb = _mm1(x, W1, deg16)
    aa, ab = _agg_kernel(xa, xb, src, dst, edge_attr)
    xa, xb = _mid(aa, ab, xa, xb, deg16, b1.reshape(1, D), W2)
    aa, ab = _agg_kernel(xa, xb, src, dst, edge_attr)
    xa, xb = _mid(aa, ab, xa, xb, deg16, b2.reshape(1, D), W3)
    aa, ab = _agg_kernel(xa, xb, src, dst, edge_attr)
    return _final(aa, ab, xa, xb, deg16, b3.reshape(1, D),
                  batch.reshape(GRID, 1, R), Wl1, bl1.reshape(1, D),
                  Wl2, bl2.reshape(1, DOUT))


# speedup vs baseline: 6.9308x; 1.2293x over previous
import functools

import jax
import jax.numpy as jnp
from jax import lax
from jax.experimental import pallas as pl
from jax.experimental.pallas import tpu as pltpu
from jax.experimental.pallas import tpu_sc as plsc

N = 10000
NP = 10240
E = 160000
EC = 128
NCH = 1280
EP = NCH * EC
D = 256
H = 128
NG = 128
DOUT = 128
NSUB = 16
RPS = NP // NSUB
CPS = NCH // NSUB
CPW = NCH // (2 * NSUB)

_MESH = plsc.VectorSubcoreMesh(core_axis_name="c", subcore_axis_name="s",
                               num_cores=2, num_subcores=NSUB)


def _zero_rows(rows):
    zero = jnp.zeros((16,), jnp.float32)

    def zrow(i, carry):
        for f in range(H // 16):
            rows[i, pl.ds(f * 16, 16)] = zero
        return carry

    lax.fori_loop(0, EC, zrow, 0)


def _zero_acc(rows, acc_sp, s):
    for k in range(RPS // EC):
        pltpu.sync_copy(rows, acc_sp.at[pl.ds(s * RPS + k * EC, EC)])


def _scale_rows(rows, wv):
    def mul(g, carry):
        w16 = wv[pl.ds(g * 16, 16)]
        base16 = g * 16
        for t in range(16):
            wb = jnp.broadcast_to(w16[t], (16,))
            j = base16 + t
            for f in range(H // 16):
                rows[j, pl.ds(f * 16, 16)] = rows[j, pl.ds(f * 16, 16)] * wb
        return carry

    lax.fori_loop(0, EC // 16, mul, 0)


def _fill_rows(rows, wv):
    def fill(g, carry):
        w16 = wv[pl.ds(g * 16, 16)]
        base16 = g * 16
        for t in range(16):
            wb = jnp.broadcast_to(w16[t], (16,))
            j = base16 + t
            for f in range(H // 16):
                rows[j, pl.ds(f * 16, 16)] = wb
        return carry

    lax.fori_loop(0, EC // 16, fill, 0)


def _make_agg(interpret=False):
    return functools.partial(
        pl.kernel,
        out_type=(
            jax.ShapeDtypeStruct((NP, H), jnp.float32),
            jax.ShapeDtypeStruct((NP, H), jnp.float32),
        ),
        mesh=_MESH,
        scratch_types=[
            pltpu.VMEM((CPS, EC), jnp.int32),
            pltpu.VMEM((EC,), jnp.int32),
            pltpu.VMEM((EC,), jnp.int32),
            pltpu.VMEM((EC,), jnp.float32),
            pltpu.VMEM((EC,), jnp.float32),
            pltpu.VMEM((EC, H), jnp.float32),
            pltpu.VMEM((EC, H), jnp.float32),
            pltpu.VMEM((EC,), jnp.int32),
            pltpu.VMEM_SHARED((NP, H), jnp.float32),
            pltpu.SemaphoreType.DMA,
            pltpu.SemaphoreType.DMA,
            pltpu.SemaphoreType.DMA,
            pltpu.SemaphoreType.DMA,
        ],
        interpret=interpret,
    )(_agg_body)


def _agg_body(xa_hbm, xb_hbm, src2_hbm, dst2_hbm, w2_hbm, flag_hbm,
              oa_hbm, ob_hbm,
              didx_v, sidx0, sidx1, wv0, wv1, rows0, rows1, flag_v, acc_sp,
              sem0, sem1, isem0, isem1):
    c = lax.axis_index("c")
    s = lax.axis_index("s")

    _zero_rows(rows0)
    _zero_acc(rows0, acc_sp, s)
    pltpu.sync_copy(flag_hbm, flag_v)
    isdeg = flag_v[pl.ds(0, 16)][0] > 0

    @pl.when(isdeg)
    def _():
        base = (s * 2 + c) * CPW
        pltpu.sync_copy(dst2_hbm.at[pl.ds(base, CPW)],
                        didx_v.at[pl.ds(0, CPW)])
        plsc.subcore_barrier()
        pltpu.sync_copy(w2_hbm.at[base], wv0)
        pltpu.sync_copy(w2_hbm.at[base + 1], wv1)

        def sstart(ci, rows, sem):
            pltpu.async_copy(rows, acc_sp.at[didx_v.at[ci]], sem, add=True)

        def swait(ci, rows, sem):
            pltpu.make_async_copy(rows, acc_sp.at[didx_v.at[ci]], sem).wait()

        last = CPW // 2 - 1

        def dpair(i2, carry):
            c0 = 2 * i2

            @pl.when(i2 > 0)
            def _():
                pltpu.make_async_copy(w2_hbm.at[base], wv0, isem0).wait()
                swait(c0 - 2, rows0, sem0)

            _fill_rows(rows0, wv0)
            sstart(c0, rows0, sem0)

            @pl.when(i2 < last)
            def _():
                pltpu.async_copy(w2_hbm.at[base + c0 + 2], wv0, isem0)

            @pl.when(i2 > 0)
            def _():
                pltpu.make_async_copy(w2_hbm.at[base], wv1, isem1).wait()
                swait(c0 - 1, rows1, sem1)

            _fill_rows(rows1, wv1)
            sstart(c0 + 1, rows1, sem1)

            @pl.when(i2 < last)
            def _():
                pltpu.async_copy(w2_hbm.at[base + c0 + 3], wv1, isem1)

            return carry

        lax.fori_loop(0, CPW // 2, dpair, 0)
        swait(CPW - 2, rows0, sem0)
        swait(CPW - 1, rows1, sem1)

    @pl.when(jnp.logical_not(isdeg))
    def _():
        base = s * CPS
        pltpu.sync_copy(dst2_hbm.at[pl.ds(base, CPS)], didx_v)
        plsc.subcore_barrier()

        def run(x_hbm):
            def gstart(si, rows, sem):
                pltpu.async_copy(x_hbm.at[si], rows, sem)

            def gwait(si, rows, sem):
                pltpu.make_async_copy(x_hbm.at[si], rows, sem).wait()

            def istart(ci, si, wv, isem):
                pltpu.async_copy(src2_hbm.at[base + ci], si, isem)
                pltpu.async_copy(w2_hbm.at[base + ci], wv, isem)

            def iwait(si, wv, isem):
                pltpu.make_async_copy(src2_hbm.at[base], si, isem).wait()
                pltpu.make_async_copy(w2_hbm.at[base], wv, isem).wait()

            def process(ci, rows, wv):
                _scale_rows(rows, wv)
                pltpu.sync_copy(rows, acc_sp.at[didx_v.at[ci]], add=True)

            pltpu.sync_copy(src2_hbm.at[base], sidx0)
            pltpu.sync_copy(w2_hbm.at[base], wv0)
            gstart(sidx0, rows0, sem0)
            istart(1, sidx1, wv1, isem1)

            last = CPS // 2 - 1

            def pair(i2, carry):
                c0 = 2 * i2
                gwait(sidx0, rows0, sem0)
                iwait(sidx1, wv1, isem1)
                gstart(sidx1, rows1, sem1)
                process(c0, rows0, wv0)

                @pl.when(i2 < last)
                def _():
                    istart(c0 + 2, sidx0, wv0, isem0)

                gwait(sidx1, rows1, sem1)

                @pl.when(i2 < last)
                def _():
                    iwait(sidx0, wv0, isem0)
                    gstart(sidx0, rows0, sem0)
                    istart(c0 + 3, sidx1, wv1, isem1)

                process(c0 + 1, rows1, wv1)
                return carry

            lax.fori_loop(0, CPS // 2, pair, 0)

        @pl.when(c == 0)
        def _():
            run(xa_hbm)

        @pl.when(c == 1)
        def _():
            run(xb_hbm)

    plsc.subcore_barrier()

    @pl.when(c == 0)
    def _():
        pltpu.sync_copy(acc_sp.at[pl.ds(s * RPS, RPS)],
                        oa_hbm.at[pl.ds(s * RPS, RPS)])

    @pl.when(c == 1)
    def _():
        pltpu.sync_copy(acc_sp.at[pl.ds(s * RPS, RPS)],
                        ob_hbm.at[pl.ds(s * RPS, RPS)])


R = 1024
GRID = NP // R


def _dinv_from(dega_ref, degb_ref):
    deg = dega_ref[...][:, :1] + degb_ref[...][:, :1] + 1.0
    return jnp.where(deg > 0, lax.rsqrt(deg), 0.0)


def _mm1_body(x_ref, w_ref, dega_ref, degb_ref, xa_ref, xb_ref):
    dinv = _dinv_from(dega_ref, degb_ref)
    xw = jnp.dot(x_ref[...], w_ref[...], preferred_element_type=jnp.float32)
    xws = xw * dinv
    xa_ref[...] = xws[:, :H]
    xb_ref[...] = xws[:, H:]


def _mm1(x, W, dega, degb):
    return pl.pallas_call(
        _mm1_body,
        grid=(GRID,),
        in_specs=[
            pl.BlockSpec((R, D), lambda i: (i, 0)),
            pl.BlockSpec((D, D), lambda i: (0, 0)),
            pl.BlockSpec((R, H), lambda i: (i, 0)),
            pl.BlockSpec((R, H), lambda i: (i, 0)),
        ],
        out_specs=[
            pl.BlockSpec((R, H), lambda i: (i, 0)),
            pl.BlockSpec((R, H), lambda i: (i, 0)),
        ],
        out_shape=[jax.ShapeDtypeStruct((NP, H), jnp.float32)] * 2,
    )(x, W, dega, degb)


def _mid_body(aa_ref, ab_ref, xa_ref, xb_ref, dega_ref, degb_ref, b_ref,
              w_ref, oa_ref, ob_ref):
    dinv = _dinv_from(dega_ref, degb_ref)
    acc = jnp.concatenate([aa_ref[...], ab_ref[...]], axis=1)
    xws = jnp.concatenate([xa_ref[...], xb_ref[...]], axis=1)
    h = jnp.maximum(dinv * (acc + xws) + b_ref[...], 0.0)
    hw = jnp.dot(h, w_ref[...], preferred_element_type=jnp.float32)
    hws = hw * dinv
    oa_ref[...] = hws[:, :H]
    ob_ref[...] = hws[:, H:]


def _mid(aa, ab, xa, xb, dega, degb, b, W):
    return pl.pallas_call(
        _mid_body,
        grid=(GRID,),
        in_specs=[
            pl.BlockSpec((R, H), lambda i: (i, 0)),
            pl.BlockSpec((R, H), lambda i: (i, 0)),
            pl.BlockSpec((R, H), lambda i: (i, 0)),
            pl.BlockSpec((R, H), lambda i: (i, 0)),
            pl.BlockSpec((R, H), lambda i: (i, 0)),
            pl.BlockSpec((R, H), lambda i: (i, 0)),
            pl.BlockSpec((1, D), lambda i: (0, 0)),
            pl.BlockSpec((D, D), lambda i: (0, 0)),
        ],
        out_specs=[
            pl.BlockSpec((R, H), lambda i: (i, 0)),
            pl.BlockSpec((R, H), lambda i: (i, 0)),
        ],
        out_shape=[jax.ShapeDtypeStruct((NP, H), jnp.float32)] * 2,
    )(aa, ab, xa, xb, dega, degb, b, W)


def _final_body(aa_ref, ab_ref, xa_ref, xb_ref, dega_ref, degb_ref, b3_ref,
                batch_ref, wl1_ref, bl1_ref, wl2_ref, bl2_ref, out_ref,
                sums, cnt):
    i = pl.program_id(0)
    dinv = _dinv_from(dega_ref, degb_ref)
    acc = jnp.concatenate([aa_ref[...], ab_ref[...]], axis=1)
    xws = jnp.concatenate([xa_ref[...], xb_ref[...]], axis=1)
    h3 = jnp.maximum(dinv * (acc + xws) + b3_ref[...], 0.0)
    seg = batch_ref[0, 0, :]
    onehot = (seg[:, None]
              == lax.broadcasted_iota(jnp.int32, (1, NG), 1)).astype(jnp.float32)
    blk_sums = lax.dot_general(onehot, h3, (((0,), (0,)), ((), ())),
                               preferred_element_type=jnp.float32)
    ones = jnp.ones((R, 1), jnp.float32)
    blk_cnt = lax.dot_general(onehot, ones, (((0,), (0,)), ((), ())),
                              preferred_element_type=jnp.float32)

    @pl.when(i == 0)
    def _():
        sums[...] = jnp.zeros_like(sums)
        cnt[...] = jnp.zeros_like(cnt)

    sums[...] += blk_sums
    cnt[...] += blk_cnt

    @pl.when(i == pl.num_programs(0) - 1)
    def _():
        pooled = sums[...] / jnp.maximum(cnt[...], 1.0)
        l1 = jnp.maximum(
            jnp.dot(pooled, wl1_ref[...], preferred_element_type=jnp.float32)
            + bl1_ref[...], 0.0)
        o = (jnp.dot(l1, wl2_ref[...], preferred_element_type=jnp.float32)
             + bl2_ref[...])
        m = jnp.max(o, axis=-1, keepdims=True)
        lse = m + jnp.log(jnp.sum(jnp.exp(o - m), axis=-1, keepdims=True))
        out_ref[...] = o - lse


def _final(aa, ab, xa, xb, dega, degb, b3, batch3, Wl1, bl1, Wl2, bl2):
    return pl.pallas_call(
        _final_body,
        grid=(GRID,),
        in_specs=[
            pl.BlockSpec((R, H), lambda i: (i, 0)),
            pl.BlockSpec((R, H), lambda i: (i, 0)),
            pl.BlockSpec((R, H), lambda i: (i, 0)),
            pl.BlockSpec((R, H), lambda i: (i, 0)),
            pl.BlockSpec((R, H), lambda i: (i, 0)),
            pl.BlockSpec((R, H), lambda i: (i, 0)),
            pl.BlockSpec((1, D), lambda i: (0, 0)),
            pl.BlockSpec((1, 1, R), lambda i: (i, 0, 0)),
            pl.BlockSpec((D, D), lambda i: (0, 0)),
            pl.BlockSpec((1, D), lambda i: (0, 0)),
            pl.BlockSpec((D, DOUT), lambda i: (0, 0)),
            pl.BlockSpec((1, DOUT), lambda i: (0, 0)),
        ],
        out_specs=pl.BlockSpec((NG, DOUT), lambda i: (0, 0)),
        out_shape=jax.ShapeDtypeStruct((NG, DOUT), jnp.float32),
        scratch_shapes=[
            pltpu.VMEM((NG, D), jnp.float32),
            pltpu.VMEM((NG, 1), jnp.float32),
        ],
    )(aa, ab, xa, xb, dega, degb, b3, batch3, Wl1, bl1, Wl2, bl2)


_agg_kernel = _make_agg()


def kernel(x, edge_index, edge_attr, batch, W1, b1, W2, b2, W3, b3,
           Wl1, bl1, Wl2, bl2):
    src = edge_index[0]
    dst = edge_index[1]
    pad = EP - E
    src2 = jnp.pad(src, (0, pad)).reshape(NCH, EC)
    dst2 = jnp.pad(dst, (0, pad)).reshape(NCH, EC)
    w2 = jnp.pad(edge_attr, (0, pad)).reshape(NCH, EC)
    x = jnp.pad(x, ((0, NP - N), (0, 0)))
    batch = jnp.pad(batch, (0, NP - N), constant_values=NG)
    dummy = jnp.zeros((NP, H), jnp.float32)
    flag1 = jnp.ones((EC,), jnp.int32)
    flag0 = jnp.zeros((EC,), jnp.int32)
    dega, degb = _agg_kernel(dummy, dummy, src2, dst2, w2, flag1)
    xa, xb = _mm1(x, W1, dega, degb)
    aa, ab = _agg_kernel(xa, xb, src2, dst2, w2, flag0)
    xa, xb = _mid(aa, ab, xa, xb, dega, degb, b1.reshape(1, D), W2)
    aa, ab = _agg_kernel(xa, xb, src2, dst2, w2, flag0)
    xa, xb = _mid(aa, ab, xa, xb, dega, degb, b2.reshape(1, D), W3)
    aa, ab = _agg_kernel(xa, xb, src2, dst2, w2, flag0)
    return _final(aa, ab, xa, xb, dega, degb, b3.reshape(1, D),
                  batch.reshape(GRID, 1, R), Wl1, bl1.reshape(1, D),
                  Wl2, bl2.reshape(1, DOUT))
